# Initial kernel scaffold; baseline (speedup 1.0000x reference)
import jax
import jax.numpy as jnp
from jax.experimental import pallas as pl


def kernel(input_, weight):
    raise NotImplementedError("write your pallas kernel here")


# speedup vs baseline: 1.7942x; 1.7942x over previous
import functools

import jax
import jax.numpy as jnp
from jax import lax
from jax.experimental import pallas as pl
from jax.experimental.pallas import tpu as pltpu
from jax.experimental.pallas import tpu_sc as plsc

D = 64
B = 16384 * 50
NC, NS = 2, 16
NW = NC * NS
B_PER_W = B // NW
CH = 512
NCHUNK = B_PER_W // CH

_mesh = plsc.VectorSubcoreMesh(core_axis_name="c", subcore_axis_name="s")


@functools.partial(
    pl.kernel,
    mesh=_mesh,
    out_type=jax.ShapeDtypeStruct((B, D), jnp.float32),
    scratch_types=[
        pltpu.VMEM((CH,), jnp.int32),
        pltpu.VMEM((CH, D), jnp.float32),
        pltpu.SemaphoreType.DMA,
    ],
    compiler_params=pltpu.CompilerParams(use_tc_tiling_on_sc=False),
)
def _emb_lookup(table_hbm, idx_hbm, out_hbm, idx_v, rows_v, sem):
    wid = lax.axis_index("s") * NC + lax.axis_index("c")
    base = wid * B_PER_W

    def body(g, carry):
        off = base + g * CH
        pltpu.sync_copy(idx_hbm.at[pl.ds(off, CH)], idx_v)
        pltpu.async_copy(table_hbm.at[idx_v], rows_v, sem).wait()
        pltpu.sync_copy(rows_v, out_hbm.at[pl.ds(off, CH)])
        return carry

    lax.fori_loop(0, NCHUNK, body, 0)


def kernel(input_, weight):
    bsz, hist = input_.shape
    idx = input_.reshape(-1).astype(jnp.int32)
    out = _emb_lookup(weight, idx)
    return out.reshape(bsz, hist, D)


# speedup vs baseline: 1.8680x; 1.0412x over previous
import functools

import jax
import jax.numpy as jnp
from jax import lax
from jax.experimental import pallas as pl
from jax.experimental.pallas import tpu as pltpu
from jax.experimental.pallas import tpu_sc as plsc

D = 64
B = 16384 * 50
NC, NS = 2, 16
NW = NC * NS
B_PER_W = B // NW
CH = 512
NBUF = 2
NCHUNK = B_PER_W // CH

_mesh = plsc.VectorSubcoreMesh(core_axis_name="c", subcore_axis_name="s")


@functools.partial(
    pl.kernel,
    mesh=_mesh,
    out_type=jax.ShapeDtypeStruct((B, D), jnp.float32),
    scratch_types=[
        pltpu.VMEM((CH,), jnp.int32),
        pltpu.VMEM((CH,), jnp.int32),
        pltpu.VMEM((CH, D), jnp.float32),
        pltpu.VMEM((CH, D), jnp.float32),
        pltpu.SemaphoreType.DMA,
        pltpu.SemaphoreType.DMA,
        pltpu.SemaphoreType.DMA,
        pltpu.SemaphoreType.DMA,
        pltpu.SemaphoreType.DMA,
        pltpu.SemaphoreType.DMA,
    ],
    compiler_params=pltpu.CompilerParams(use_tc_tiling_on_sc=False),
)
def _emb_lookup(table_hbm, idx_hbm, out_hbm,
                idx_v0, idx_v1, rows_v0, rows_v1,
                isem0, isem1, gsem0, gsem1, ssem0, ssem1):
    idx_v = (idx_v0, idx_v1)
    rows_v = (rows_v0, rows_v1)
    isems = (isem0, isem1)
    gsems = (gsem0, gsem1)
    ssems = (ssem0, ssem1)

    wid = lax.axis_index("s") * NC + lax.axis_index("c")
    base = wid * B_PER_W

    for b in range(NBUF):
        pltpu.async_copy(
            idx_hbm.at[pl.ds(base + b * CH, CH)], idx_v[b], isems[b])

    def body(i, carry):
        g0 = i * NBUF
        for b in range(NBUF):
            off = base + (g0 + b) * CH

            @pl.when(i > 0)
            def _wait_store(b=b, off=off):
                pltpu.make_async_copy(
                    rows_v[b], out_hbm.at[pl.ds(base, CH)], ssems[b]).wait()

            pltpu.make_async_copy(
                idx_hbm.at[pl.ds(off, CH)], idx_v[b], isems[b]).wait()
            pltpu.async_copy(table_hbm.at[idx_v[b]], rows_v[b], gsems[b])

        for b in range(NBUF):
            off = base + (g0 + b) * CH
            pltpu.make_async_copy(
                table_hbm.at[idx_v[b]], rows_v[b], gsems[b]).wait()

            @pl.when(g0 + b + NBUF < NCHUNK)
            def _prefetch_idx(b=b, off=off):
                pltpu.async_copy(
                    idx_hbm.at[pl.ds(off + NBUF * CH, CH)], idx_v[b], isems[b])

            pltpu.async_copy(rows_v[b], out_hbm.at[pl.ds(off, CH)], ssems[b])
        return carry

    lax.fori_loop(0, NCHUNK // NBUF, body, 0)

    for b in range(NBUF):
        pltpu.make_async_copy(
            rows_v[b], out_hbm.at[pl.ds(base, CH)], ssems[b]).wait()


def kernel(input_, weight):
    bsz, hist = input_.shape
    idx = input_.reshape(-1).astype(jnp.int32)
    out = _emb_lookup(weight, idx)
    return out.reshape(bsz, hist, D)
